# baseline (speedup 1.0000x reference)
import functools

import jax
import jax.numpy as jnp
from jax import lax
from jax.experimental import pallas as pl
from jax.experimental.pallas import tpu as pltpu
from jax.experimental.pallas import tpu_sc as plsc

_SIZE = (2048, 2048)
_BLK = 256
_NREG = 64
_REG_AREA = _BLK * _BLK
_NB = 8
_REP = 4
_HALVES = 2
_BANDS_PER_HALF = _NB // _HALVES
_QUARTS = 32 // _BANDS_PER_HALF
_ROWS_PER_W = _BLK // _QUARTS


def _sc_body(half, ids_hbm, wr_hbm, wi_hbm, or_hbm, oi_hbm,
             ids_v, wr_v, wi_v, row_r, row_i, sem):
    wid = lax.axis_index("s") * 2 + lax.axis_index("c")
    band = wid % _BANDS_PER_HALF
    quarter = wid // _BANDS_PER_HALF
    gband = half * _BANDS_PER_HALF + band

    pltpu.sync_copy(ids_hbm, ids_v)
    pltpu.sync_copy(wr_hbm, wr_v)
    pltpu.sync_copy(wi_hbm, wi_v)

    ids16 = ids_v[pl.ds(gband * 8, 16)]
    for s in range(8):
        gid = ids16[s]
        wr16 = wr_v[pl.ds(gid, 16)]
        wi16 = wi_v[pl.ds(gid, 16)]
        vr16 = 4.0 / (1.0 + jnp.exp(-wr16)) + 1.0
        vi16 = 1.0 / (1.0 + jnp.exp(-wi16))
        bro_r = jnp.full((16,), vr16[0], jnp.float32)
        bro_i = jnp.full((16,), vi16[0], jnp.float32)
        for k in range(16):
            for rr in range(_REP):
                row_r[rr, pl.ds(s * _BLK + k * 16, 16)] = bro_r
                row_i[rr, pl.ds(s * _BLK + k * 16, 16)] = bro_i

    y0 = band * _BLK + quarter * _ROWS_PER_W
    descs = []
    for r in range(_ROWS_PER_W // _REP):
        descs.append(pltpu.async_copy(
            row_r, or_hbm.at[pl.ds(y0 + r * _REP, _REP)], sem))
        descs.append(pltpu.async_copy(
            row_i, oi_hbm.at[pl.ds(y0 + r * _REP, _REP)], sem))
    for d in descs:
        d.wait()


def kernel(weight_real, weight_imag, gathering_indices, scattering_indices,
           field_real, field_imag):
    region_ids = gathering_indices.reshape(_NREG, _REG_AREA)[:, 0]
    bases = scattering_indices.reshape(_NREG, _REG_AREA)[:, 0]
    slots = (bases // (_BLK * _SIZE[1])) * 8 + (bases % _SIZE[1]) // _BLK
    slot_ids = jnp.zeros((2 * _NREG,), region_ids.dtype).at[slots].set(region_ids)

    half_rows = _SIZE[0] // _HALVES
    halves = []
    for h in range(_HALVES):
        run = functools.partial(
            pl.kernel,
            out_type=[
                jax.ShapeDtypeStruct((half_rows, _SIZE[1]), jnp.float32),
                jax.ShapeDtypeStruct((half_rows, _SIZE[1]), jnp.float32),
            ],
            mesh=plsc.VectorSubcoreMesh(core_axis_name="c", subcore_axis_name="s"),
            scratch_types=[
                pltpu.VMEM((2 * _NREG,), jnp.int32),
                pltpu.VMEM((_SIZE[0],), jnp.float32),
                pltpu.VMEM((_SIZE[0],), jnp.float32),
                pltpu.VMEM((_REP, _SIZE[1]), jnp.float32),
                pltpu.VMEM((_REP, _SIZE[1]), jnp.float32),
                pltpu.SemaphoreType.DMA,
            ],
            name=f"sc_fill_h{h}",
        )(functools.partial(_sc_body, h))
        halves.append(run(slot_ids, weight_real, weight_imag))

    out = jnp.zeros(_SIZE, jnp.complex64)
    for h, (fr, fi) in enumerate(halves):
        out = jax.lax.dynamic_update_slice(
            out, jax.lax.complex(fr, fi), (h * half_rows, 0))
    return out
